# baseline (speedup 1.0000x reference)
import functools

import jax
import jax.numpy as jnp
from jax import lax
from jax.experimental import pallas as pl
from jax.experimental.pallas import tpu as pltpu
from jax.experimental.pallas import tpu_sc as plsc

N = 10000
D = 128
E = 320000

NC = 2
NS = 16
NW = NC * NS
K = 128
NCHUNK = 80
KA = 80
NCHA = 128
NBUF = 4
NIB = 8
EPW = 10240
E_PAD = NW * K * NCHUNK
N_PAD = 10240
RPT = N_PAD // NS

_mesh = plsc.VectorSubcoreMesh(core_axis_name="c", subcore_axis_name="s")

_f32 = jnp.float32
_i32 = jnp.int32


@functools.partial(
    pl.kernel,
    out_type=jax.ShapeDtypeStruct((NC, N_PAD), _f32),
    mesh=_mesh,
    scratch_types=[
        pltpu.VMEM((NCHUNK, K), _i32),
        pltpu.VMEM((NCHUNK, K), _f32),
        pltpu.VMEM((K,), _f32),
        pltpu.VMEM_SHARED((N_PAD,), _f32),
        pltpu.SemaphoreType.DMA,
    ],
)
def _deg(col_hbm, w_hbm, out_hbm, colall, wall, zbuf, degacc, ssem):
    c = lax.axis_index("c")
    s = lax.axis_index("s")
    wid = c * NS + s
    zero16 = jnp.zeros((16,), _f32)
    for g in range(K // 16):
        zbuf[pl.ds(g * 16, 16)] = zero16
    for i in range(RPT // K):
        pltpu.sync_copy(zbuf, degacc.at[pl.ds(s * RPT + i * K, K)])
    pltpu.sync_copy(col_hbm.at[pl.ds(wid * NCHUNK, NCHUNK)], colall)
    pltpu.sync_copy(w_hbm.at[pl.ds(wid * NCHUNK, NCHUNK)], wall)
    plsc.subcore_barrier()

    def fire(ci, carry):
        pltpu.async_copy(wall.at[ci], degacc.at[colall.at[ci]], ssem, add=True)
        return carry

    lax.fori_loop(0, NCHUNK, fire, 0)

    def drain(ci, carry):
        pltpu.make_async_copy(wall.at[0], degacc.at[colall.at[0]], ssem).wait()
        return carry

    lax.fori_loop(0, NCHUNK, drain, 0)
    plsc.subcore_barrier()
    pltpu.sync_copy(degacc.at[pl.ds(s * RPT, RPT)],
                    out_hbm.at[c, pl.ds(s * RPT, RPT)])


@functools.partial(
    pl.kernel,
    out_type=jax.ShapeDtypeStruct((NC, N_PAD, D), _f32),
    mesh=_mesh,
    scratch_types=[
        [pltpu.VMEM((KA,), _i32)] * NIB,
        [pltpu.VMEM((KA,), _i32)] * NIB,
        [pltpu.VMEM((KA,), _f32)] * NIB,
        [pltpu.VMEM((KA, D), _f32)] * NBUF,
        pltpu.VMEM_SHARED((N_PAD, D), _f32),
        [pltpu.SemaphoreType.DMA] * NBUF,
        [pltpu.SemaphoreType.DMA] * NBUF,
        [pltpu.SemaphoreType.DMA] * NIB,
    ],
)
def _agg(xs_hbm, row_hbm, col_hbm, w_hbm, out_hbm, rowbs, colbs, wbs, rbufs,
         acc, gsems, ssems, isems):
    c = lax.axis_index("c")
    s = lax.axis_index("s")
    wid = c * NS + s
    base0 = wid * EPW
    zero16 = jnp.zeros((16,), _f32)
    r0 = rbufs[0]

    def zrow(e, carry):
        for j in range(D // 16):
            r0[e, pl.ds(j * 16, 16)] = zero16
        return carry

    lax.fori_loop(0, KA, zrow, 0)
    for i in range(RPT // KA):
        pltpu.sync_copy(r0, acc.at[pl.ds(s * RPT + i * KA, KA)])

    def idx_dma(ci, m):
        base = base0 + ci * KA
        pltpu.async_copy(row_hbm.at[pl.ds(base, KA)], rowbs[m], isems[m])
        pltpu.async_copy(col_hbm.at[pl.ds(base, KA)], colbs[m], isems[m])
        pltpu.async_copy(w_hbm.at[pl.ds(base, KA)], wbs[m], isems[m])

    def idx_wait(m):
        pltpu.make_async_copy(row_hbm.at[pl.ds(0, KA)], rowbs[m],
                              isems[m]).wait()
        pltpu.make_async_copy(col_hbm.at[pl.ds(0, KA)], colbs[m],
                              isems[m]).wait()
        pltpu.make_async_copy(w_hbm.at[pl.ds(0, KA)], wbs[m], isems[m]).wait()

    for m0 in range(6):
        idx_dma(m0, m0)
    plsc.subcore_barrier()
    idx_wait(0)
    idx_wait(1)
    for pb in (0, 1):
        pltpu.async_copy(xs_hbm.at[rowbs[pb].at[pl.ds(0, KA // 2)]],
                         rbufs[pb].at[pl.ds(0, KA // 2)], gsems[pb])
        pltpu.async_copy(xs_hbm.at[rowbs[pb].at[pl.ds(KA // 2, KA // 2)]],
                         rbufs[pb].at[pl.ds(KA // 2, KA // 2)], gsems[pb])

    def rnd(t, carry):
        for off in range(NIB):
            ci = NIB * t + off
            b = off % NBUF
            m = off
            rb = rbufs[b]
            pltpu.make_async_copy(xs_hbm.at[rowbs[0]], rb, gsems[b]).wait()

            def grp(g, inner):
                wv = wbs[m][pl.ds(g * 16, 16)]
                for k in range(16):
                    e = g * 16 + k
                    ws = wv.at[jnp.full((16,), k, _i32)].get(
                        mode="promise_in_bounds")
                    for j in range(D // 16):
                        sl = pl.ds(j * 16, 16)
                        rb[e, sl] = rb[e, sl] * ws
                return inner

            lax.fori_loop(0, KA // 16, grp, 0)
            pltpu.async_copy(rb, acc.at[colbs[m]], ssems[b], add=True)

            q = (b + 2) % NBUF
            mj = (off + 2) % NIB
            mi = (off + 6) % NIB
            cj = ci + 2
            cn = ci + 6

            @pl.when(jnp.logical_and(cj >= NBUF, cj < NCHA))
            def _():
                pltpu.make_async_copy(rbufs[q], acc.at[colbs[0]],
                                      ssems[q]).wait()

            @pl.when(cn < NCHA)
            def _():
                idx_dma(cn, mi)

            @pl.when(cj < NCHA)
            def _():
                idx_wait(mj)
                pltpu.async_copy(xs_hbm.at[rowbs[mj].at[pl.ds(0, KA // 2)]],
                                 rbufs[q].at[pl.ds(0, KA // 2)], gsems[q])
                pltpu.async_copy(
                    xs_hbm.at[rowbs[mj].at[pl.ds(KA // 2, KA // 2)]],
                    rbufs[q].at[pl.ds(KA // 2, KA // 2)], gsems[q])
        return carry

    lax.fori_loop(0, NCHA // NIB, rnd, 0)
    for b in range(NBUF):
        pltpu.make_async_copy(rbufs[b], acc.at[colbs[0]], ssems[b]).wait()
    plsc.subcore_barrier()
    pltpu.sync_copy(acc.at[pl.ds(s * RPT, RPT)],
                    out_hbm.at[c, pl.ds(s * RPT, RPT)])


def _mm1_body(x_ref, w1_ref, degp_ref, xs_ref, dinv_ref):
    deg = degp_ref[0, :N, :] + degp_ref[1, :N, :] + 1.0
    dinv = lax.rsqrt(deg)
    xw = jnp.dot(x_ref[...], w1_ref[...], preferred_element_type=_f32)
    xs_ref[...] = xw * dinv
    dinv_ref[...] = dinv


_mm1 = pl.pallas_call(
    _mm1_body,
    out_shape=(jax.ShapeDtypeStruct((N, D), _f32),
               jax.ShapeDtypeStruct((N, 1), _f32)),
)


def _mid_body(agg_ref, xs1_ref, dinv_ref, b1_ref, w2_ref, xs2_ref):
    ssum = agg_ref[0, :N, :] + agg_ref[1, :N, :] + xs1_ref[...]
    dinv = dinv_ref[...]
    h = jnp.maximum(ssum * dinv + b1_ref[...][None, :], 0.0)
    xs2_ref[...] = jnp.dot(h, w2_ref[...], preferred_element_type=_f32) * dinv


_mid = pl.pallas_call(
    _mid_body,
    out_shape=jax.ShapeDtypeStruct((N, D), _f32),
)


def _out_body(agg_ref, xs2_ref, dinv_ref, b2_ref, out_ref):
    ssum = agg_ref[0, :N, :] + agg_ref[1, :N, :] + xs2_ref[...]
    out_ref[...] = ssum * dinv_ref[...] + b2_ref[...][None, :]


_out = pl.pallas_call(
    _out_body,
    out_shape=jax.ShapeDtypeStruct((N, D), _f32),
)


def kernel(x, edge_index, edge_attr, W1, b1, W2, b2):
    pad = E_PAD - E
    spread = jnp.arange(pad, dtype=_i32) % N
    rowp = jnp.concatenate([edge_index[0], spread])
    colp = jnp.concatenate([edge_index[1], spread])
    wp = jnp.concatenate([edge_attr, jnp.zeros((pad,), _f32)])
    col2 = colp.reshape(E_PAD // K, K)
    w2 = wp.reshape(E_PAD // K, K)

    degp = _deg(col2, w2).reshape(NC, N_PAD, 1)
    xs1, dinv = _mm1(x, W1, degp)
    agg1 = _agg(xs1, rowp, colp, wp)
    xs2 = _mid(agg1, xs1, dinv, b1, W2)
    agg2 = _agg(xs2, rowp, colp, wp)
    return _out(agg2, xs2, dinv, b2)


# speedup vs baseline: 1.0034x; 1.0034x over previous
import functools

import jax
import jax.numpy as jnp
from jax import lax
from jax.experimental import pallas as pl
from jax.experimental.pallas import tpu as pltpu
from jax.experimental.pallas import tpu_sc as plsc

N = 10000
D = 128
E = 320000

NC = 2
NS = 16
NW = NC * NS
K = 128
NCHUNK = 80
KA = 80
NCHA = 128
NBUF = 4
NIB = 8
EPW = 10240
E_PAD = NW * K * NCHUNK
N_PAD = 10240
RPT = N_PAD // NS

_mesh = plsc.VectorSubcoreMesh(core_axis_name="c", subcore_axis_name="s")

_f32 = jnp.float32
_i32 = jnp.int32


@functools.partial(
    pl.kernel,
    out_type=jax.ShapeDtypeStruct((NC, N_PAD), _f32),
    mesh=_mesh,
    scratch_types=[
        pltpu.VMEM((NCHUNK, K), _i32),
        pltpu.VMEM((NCHUNK, K), _f32),
        pltpu.VMEM((K,), _f32),
        pltpu.VMEM_SHARED((N_PAD,), _f32),
        pltpu.SemaphoreType.DMA,
    ],
)
def _deg(col_hbm, w_hbm, out_hbm, colall, wall, zbuf, degacc, ssem):
    c = lax.axis_index("c")
    s = lax.axis_index("s")
    wid = c * NS + s
    zero16 = jnp.zeros((16,), _f32)
    for g in range(K // 16):
        zbuf[pl.ds(g * 16, 16)] = zero16
    for i in range(RPT // K):
        pltpu.sync_copy(zbuf, degacc.at[pl.ds(s * RPT + i * K, K)])
    pltpu.sync_copy(col_hbm.at[pl.ds(wid * NCHUNK, NCHUNK)], colall)
    pltpu.sync_copy(w_hbm.at[pl.ds(wid * NCHUNK, NCHUNK)], wall)
    plsc.subcore_barrier()

    def fire(ci, carry):
        pltpu.async_copy(wall.at[ci], degacc.at[colall.at[ci]], ssem, add=True)
        return carry

    lax.fori_loop(0, NCHUNK, fire, 0)

    def drain(ci, carry):
        pltpu.make_async_copy(wall.at[0], degacc.at[colall.at[0]], ssem).wait()
        return carry

    lax.fori_loop(0, NCHUNK, drain, 0)
    plsc.subcore_barrier()
    pltpu.sync_copy(degacc.at[pl.ds(s * RPT, RPT)],
                    out_hbm.at[c, pl.ds(s * RPT, RPT)])


@functools.partial(
    pl.kernel,
    out_type=jax.ShapeDtypeStruct((NC, N_PAD, D), _f32),
    mesh=_mesh,
    scratch_types=[
        [pltpu.VMEM((KA,), _i32)] * NIB,
        [pltpu.VMEM((KA,), _i32)] * NIB,
        [pltpu.VMEM((KA,), _f32)] * NIB,
        [pltpu.VMEM((KA, D), _f32)] * NBUF,
        pltpu.VMEM_SHARED((N_PAD, D), _f32),
        [pltpu.SemaphoreType.DMA] * NBUF,
        [pltpu.SemaphoreType.DMA] * NBUF,
        [pltpu.SemaphoreType.DMA] * NIB,
    ],
)
def _agg(xs_hbm, row_hbm, col_hbm, w_hbm, out_hbm, rowbs, colbs, wbs, rbufs,
         acc, gsems, ssems, isems):
    c = lax.axis_index("c")
    s = lax.axis_index("s")
    wid = c * NS + s
    base0 = wid * EPW
    zero16 = jnp.zeros((16,), _f32)
    r0 = rbufs[0]

    def zrow(e, carry):
        for j in range(D // 16):
            r0[e, pl.ds(j * 16, 16)] = zero16
        return carry

    lax.fori_loop(0, KA, zrow, 0)
    for i in range(RPT // KA):
        pltpu.async_copy(r0, acc.at[pl.ds(s * RPT + i * KA, KA)], gsems[0])
    for i in range(RPT // KA):
        pltpu.make_async_copy(r0, acc.at[pl.ds(s * RPT, KA)], gsems[0]).wait()

    def idx_dma(ci, m):
        base = base0 + ci * KA
        pltpu.async_copy(row_hbm.at[pl.ds(base, KA)], rowbs[m], isems[m])
        pltpu.async_copy(col_hbm.at[pl.ds(base, KA)], colbs[m], isems[m])
        pltpu.async_copy(w_hbm.at[pl.ds(base, KA)], wbs[m], isems[m])

    def idx_wait(m):
        pltpu.make_async_copy(row_hbm.at[pl.ds(0, KA)], rowbs[m],
                              isems[m]).wait()
        pltpu.make_async_copy(col_hbm.at[pl.ds(0, KA)], colbs[m],
                              isems[m]).wait()
        pltpu.make_async_copy(w_hbm.at[pl.ds(0, KA)], wbs[m], isems[m]).wait()

    for m0 in range(6):
        idx_dma(m0, m0)
    plsc.subcore_barrier()
    idx_wait(0)
    idx_wait(1)
    pltpu.async_copy(xs_hbm.at[rowbs[0]], rbufs[0], gsems[0])
    pltpu.async_copy(xs_hbm.at[rowbs[1]], rbufs[1], gsems[1])

    def rnd(t, carry):
        for off in range(NIB):
            ci = NIB * t + off
            b = off % NBUF
            m = off
            rb = rbufs[b]
            pltpu.make_async_copy(xs_hbm.at[rowbs[0]], rb, gsems[b]).wait()

            def grp(g, inner):
                wv = wbs[m][pl.ds(g * 16, 16)]
                for k in range(16):
                    e = g * 16 + k
                    ws = wv.at[jnp.full((16,), k, _i32)].get(
                        mode="promise_in_bounds")
                    for j in range(D // 16):
                        sl = pl.ds(j * 16, 16)
                        rb[e, sl] = rb[e, sl] * ws
                return inner

            lax.fori_loop(0, KA // 16, grp, 0)
            pltpu.async_copy(rb, acc.at[colbs[m]], ssems[b], add=True)

            q = (b + 2) % NBUF
            mj = (off + 2) % NIB
            mi = (off + 6) % NIB
            cj = ci + 2
            cn = ci + 6

            @pl.when(jnp.logical_and(cj >= NBUF, cj < NCHA))
            def _():
                pltpu.make_async_copy(rbufs[q], acc.at[colbs[0]],
                                      ssems[q]).wait()

            @pl.when(cn < NCHA)
            def _():
                idx_dma(cn, mi)

            @pl.when(cj < NCHA)
            def _():
                idx_wait(mj)
                pltpu.async_copy(xs_hbm.at[rowbs[mj]], rbufs[q], gsems[q])
        return carry

    lax.fori_loop(0, NCHA // NIB, rnd, 0)
    for b in range(NBUF):
        pltpu.make_async_copy(rbufs[b], acc.at[colbs[0]], ssems[b]).wait()
    plsc.subcore_barrier()
    pltpu.sync_copy(acc.at[pl.ds(s * RPT, RPT)],
                    out_hbm.at[c, pl.ds(s * RPT, RPT)])


def _mm1_body(x_ref, w1_ref, degp_ref, xs_ref, dinv_ref):
    deg = degp_ref[0, :N, :] + degp_ref[1, :N, :] + 1.0
    dinv = lax.rsqrt(deg)
    xw = jnp.dot(x_ref[...], w1_ref[...], preferred_element_type=_f32)
    xs_ref[...] = xw * dinv
    dinv_ref[...] = dinv


_mm1 = pl.pallas_call(
    _mm1_body,
    out_shape=(jax.ShapeDtypeStruct((N, D), _f32),
               jax.ShapeDtypeStruct((N, 1), _f32)),
)


def _mid_body(agg_ref, xs1_ref, dinv_ref, b1_ref, w2_ref, xs2_ref):
    ssum = agg_ref[0, :N, :] + agg_ref[1, :N, :] + xs1_ref[...]
    dinv = dinv_ref[...]
    h = jnp.maximum(ssum * dinv + b1_ref[...][None, :], 0.0)
    xs2_ref[...] = jnp.dot(h, w2_ref[...], preferred_element_type=_f32) * dinv


_mid = pl.pallas_call(
    _mid_body,
    out_shape=jax.ShapeDtypeStruct((N, D), _f32),
)


def _out_body(agg_ref, xs2_ref, dinv_ref, b2_ref, out_ref):
    ssum = agg_ref[0, :N, :] + agg_ref[1, :N, :] + xs2_ref[...]
    out_ref[...] = ssum * dinv_ref[...] + b2_ref[...][None, :]


_out = pl.pallas_call(
    _out_body,
    out_shape=jax.ShapeDtypeStruct((N, D), _f32),
)


def kernel(x, edge_index, edge_attr, W1, b1, W2, b2):
    pad = E_PAD - E
    spread = jnp.arange(pad, dtype=_i32) % N
    rowp = jnp.concatenate([edge_index[0], spread])
    colp = jnp.concatenate([edge_index[1], spread])
    wp = jnp.concatenate([edge_attr, jnp.zeros((pad,), _f32)])
    col2 = colp.reshape(E_PAD // K, K)
    w2 = wp.reshape(E_PAD // K, K)

    degp = _deg(col2, w2).reshape(NC, N_PAD, 1)
    xs1, dinv = _mm1(x, W1, degp)
    agg1 = _agg(xs1, rowp, colp, wp)
    xs2 = _mid(agg1, xs1, dinv, b1, W2)
    agg2 = _agg(xs2, rowp, colp, wp)
    return _out(agg2, xs2, dinv, b2)
